# baseline (speedup 1.0000x reference)
import functools

import jax
import jax.numpy as jnp
from jax import lax
from jax.experimental import pallas as pl
from jax.experimental.pallas import tpu as pltpu, tpu_sc as plsc

_NC = 2
_NS = 16
_NW = _NC * _NS
_L = 16

_BATCH = 16384
_DIM = 16
_BPW = _BATCH // _NW
_CHUNK = 128
_NCHUNK = _BPW // _CHUNK
_NGRP = _BPW // _L


def _lfm_body(user_ref, item_ref, w_user_ref, w_item_ref,
              w_bias_user_ref, w_bias_item_ref, bg_ref, out_ref,
              uidx_v, iidx_v, urows_v, irows_v, ubias_v, ibias_v,
              bg_v, out_v, sem):
    wid = lax.axis_index("s") * _NC + lax.axis_index("c")
    base = wid * _BPW

    pltpu.sync_copy(user_ref.at[pl.ds(wid * _NCHUNK, _NCHUNK)], uidx_v)
    pltpu.sync_copy(item_ref.at[pl.ds(wid * _NCHUNK, _NCHUNK)], iidx_v)
    pltpu.sync_copy(bg_ref, bg_v)

    copies = []
    for j in range(_NCHUNK):
        dst = pl.ds(j * _CHUNK, _CHUNK)
        copies.append(pltpu.async_copy(
            w_user_ref.at[uidx_v.at[j]], urows_v.at[dst], sem))
        copies.append(pltpu.async_copy(
            w_item_ref.at[iidx_v.at[j]], irows_v.at[dst], sem))
        copies.append(pltpu.async_copy(
            w_bias_user_ref.at[uidx_v.at[j]], ubias_v.at[dst], sem))
        copies.append(pltpu.async_copy(
            w_bias_item_ref.at[iidx_v.at[j]], ibias_v.at[dst], sem))
    for c in copies:
        c.wait()

    lanes = lax.iota(jnp.int32, _L)
    bg = bg_v[...]
    cols = [jnp.full((_L,), d, jnp.int32) for d in range(_DIM)]

    def group(g, _):
        rows = g * _L + lanes
        acc = bg + ubias_v[pl.ds(g * _L, _L)] + ibias_v[pl.ds(g * _L, _L)]
        for d in range(_DIM):
            u_d = plsc.load_gather(urows_v, [rows, cols[d]])
            v_d = plsc.load_gather(irows_v, [rows, cols[d]])
            acc = acc + u_d * v_d
        out_v[pl.ds(g * _L, _L)] = acc
        return _

    lax.fori_loop(0, _NGRP, group, 0)
    pltpu.sync_copy(out_v, out_ref.at[pl.ds(base, _BPW)])


@jax.jit
def kernel(user, item, w_user, w_item, w_bias_user, w_bias_item, bias_global):
    mesh = plsc.VectorSubcoreMesh(
        core_axis_name="c", subcore_axis_name="s",
        num_cores=_NC, num_subcores=_NS)
    lfm = functools.partial(
        pl.kernel,
        out_type=jax.ShapeDtypeStruct((_BATCH,), jnp.float32),
        mesh=mesh,
        compiler_params=pltpu.CompilerParams(
            needs_layout_passes=False, use_tc_tiling_on_sc=False),
        scratch_types=[
            pltpu.VMEM((_NCHUNK, _CHUNK), jnp.int32),
            pltpu.VMEM((_NCHUNK, _CHUNK), jnp.int32),
            pltpu.VMEM((_BPW, _DIM), jnp.float32),
            pltpu.VMEM((_BPW, _DIM), jnp.float32),
            pltpu.VMEM((_BPW,), jnp.float32),
            pltpu.VMEM((_BPW,), jnp.float32),
            pltpu.VMEM((_L,), jnp.float32),
            pltpu.VMEM((_BPW,), jnp.float32),
            pltpu.SemaphoreType.DMA,
        ],
    )(_lfm_body)
    user2 = user.reshape(_NW * _NCHUNK, _CHUNK)
    item2 = item.reshape(_NW * _NCHUNK, _CHUNK)
    bg16 = jnp.broadcast_to(bias_global, (_L,)).astype(jnp.float32)
    return lfm(user2, item2, w_user, w_item, w_bias_user, w_bias_item, bg16)
